# baseline (speedup 1.0000x reference)
import functools

import jax
import jax.numpy as jnp
from jax import lax
from jax.experimental import pallas as pl
from jax.experimental.pallas import tpu as pltpu
from jax.experimental.pallas import tpu_sc as plsc

N_VARS = 25000
N_CLAUSES = 100000
NNZ = 300000
N_GRAPHS = 16
FM = 64
QM = 64
ROUNDS = 8
LITS = 2 * N_VARS

BT = 1000
NBV = N_VARS // BT
NBC = N_CLAUSES // BT

EB = 128
NTILES = 32
CH1 = 1024
CH2 = 512
NCH1 = -(-N_CLAUSES // CH1)
NCH2 = -(-LITS // CH2)
KMAX1 = -(-NCH1 // NTILES)
KMAX2 = -(-NCH2 // NTILES)


def _make_seg_sum(w, ch, n_chunks, kmax):
    mesh = plsc.VectorSubcoreMesh(core_axis_name="c", subcore_axis_name="s")
    out_rows = n_chunks * ch
    slot = ch + 1

    @functools.partial(
        pl.kernel,
        out_type=jax.ShapeDtypeStruct((out_rows, w), jnp.float32),
        mesh=mesh,
        scratch_types=[
            pltpu.VMEM((16,), jnp.int32),
            pltpu.VMEM((EB,), jnp.int32),
            pltpu.VMEM((EB,), jnp.int32),
            pltpu.VMEM((EB, w), jnp.float32),
            pltpu.VMEM_SHARED((16 * slot, w), jnp.float32),
            pltpu.SemaphoreType.DMA,
        ],
        compiler_params=pltpu.CompilerParams(use_tc_tiling_on_sc=False,
                                             needs_layout_passes=False),
    )
    def seg(table, eidx, lidx, meta, zeros, out,
            meta_v, eidx_v, lidx_v, ebuf, acc, sem):
        wid = lax.axis_index("s") * 2 + lax.axis_index("c")
        sid = lax.axis_index("s")
        pltpu.sync_copy(meta.at[wid], meta_v)
        mv = meta_v[...]
        lanes = lax.iota(jnp.int32, 16)
        for k in range(kmax):
            chunk = wid + NTILES * k
            s = jnp.sum(jnp.where(lanes == 2 * k, mv, 0))
            e = jnp.sum(jnp.where(lanes == 2 * k + 1, mv, 0))
            nb = (e - s) // EB

            @pl.when(chunk < n_chunks)
            def _():
                pltpu.sync_copy(zeros, acc.at[pl.ds(sid * slot, slot)])

                def body(b, carry):
                    off = pl.multiple_of(s + b * EB, EB)
                    pltpu.sync_copy(eidx.at[pl.ds(off, EB)], eidx_v)
                    pltpu.sync_copy(lidx.at[pl.ds(off, EB)], lidx_v)
                    pltpu.async_copy(table.at[eidx_v], ebuf, sem).wait()
                    pltpu.sync_copy(ebuf, acc.at[lidx_v], add=True)
                    return carry

                lax.fori_loop(0, nb, body, 0)
                pltpu.sync_copy(acc.at[pl.ds(sid * slot, ch)],
                                out.at[pl.ds(chunk * ch, ch)])

    return seg


@functools.lru_cache(maxsize=None)
def _seg_sum_cached(w, ch, n_chunks, kmax):
    return _make_seg_sum(w, ch, n_chunks, kmax)


def _full(a):
    return pl.BlockSpec(a.shape, lambda i: (0,) * a.ndim)


def _rows(w, nd_extra=None):
    return pl.BlockSpec((BT, w), lambda i: (i, 0))


def _k1_body(v, nz, w1a, w1b, b1, w2, b2, lit):
    q = jnp.maximum(v[...] @ w1a[...] + nz[...] @ w1b[...] + b1[...], 0.0)
    q = q @ w2[...] + b2[...]
    lit[0] = jax.nn.softplus(q)
    lit[1] = jax.nn.softplus(-q)


def _k1(v, nz, w1a, w1b, b1, w2, b2):
    return pl.pallas_call(
        _k1_body,
        grid=(NBV,),
        in_specs=[_rows(FM), pl.BlockSpec((BT, 4), lambda i: (i, 0)),
                  _full(w1a), _full(w1b), _full(b1), _full(w2), _full(b2)],
        out_specs=pl.BlockSpec((2, BT, FM), lambda i: (0, i, 0)),
        out_shape=jax.ShapeDtypeStruct((2, N_VARS, FM), jnp.float32),
    )(v, nz, w1a, w1b, b1, w2, b2)


def _k2_body(cval, cold, ohw, w1a, w1b, b1, w2, b2, tab, cd2r, gs):
    cl = jnp.exp(-cval[...])
    h = jnp.maximum(cold[...] @ w1a[...] + (4.0 * cl) @ w1b[...] + b1[...], 0.0)
    cd = h @ w2[...] + b2[...]
    tab[:, :FM] = cl
    tab[:, FM:] = cd[:, :FM]
    cd2 = cd[:, QM:]
    cd2r[...] = cd2

    @pl.when(pl.program_id(0) == 0)
    def _():
        gs[...] = jnp.zeros_like(gs)

    gs[...] += lax.dot_general(ohw[...], cd2, (((0,), (0,)), ((), ())))


def _k2(cval, cold, ohw, w1a, w1b, b1, w2, b2):
    return pl.pallas_call(
        _k2_body,
        grid=(NBC,),
        in_specs=[_rows(FM), _rows(FM),
                  pl.BlockSpec((BT, N_GRAPHS), lambda i: (i, 0)),
                  _full(w1a), _full(w1b), _full(b1), _full(w2), _full(b2)],
        out_specs=[_rows(FM + QM), _rows(FM),
                   pl.BlockSpec((N_GRAPHS, FM), lambda i: (0, 0))],
        out_shape=[jax.ShapeDtypeStruct((N_CLAUSES, FM + QM), jnp.float32),
                   jax.ShapeDtypeStruct((N_CLAUSES, FM), jnp.float32),
                   jax.ShapeDtypeStruct((N_GRAPHS, FM), jnp.float32)],
    )(cval, cold, ohw, w1a, w1b, b1, w2, b2)


def _pn_body(x, old, ohw, gs, out, *, scale):
    xm = x[...] - ohw[...] @ gs[...]
    var = jnp.mean(xm * xm, axis=1, keepdims=True)
    xn = xm * lax.rsqrt(var + 1e-6)
    out[...] = (0.25 * xn + 0.1 * old[...]) * scale


def _pn(x, old, ohw, gs, scale):
    n = x.shape[0]
    return pl.pallas_call(
        functools.partial(_pn_body, scale=scale),
        grid=(n // BT,),
        in_specs=[_rows(FM), _rows(FM),
                  pl.BlockSpec((BT, N_GRAPHS), lambda i: (i, 0)),
                  pl.BlockSpec((N_GRAPHS, FM), lambda i: (0, 0))],
        out_specs=_rows(FM),
        out_shape=jax.ShapeDtypeStruct((n, FM), jnp.float32),
    )(x, old, ohw, gs)


def _k3_body(v, litp, litn, sp, sn, vdw, dwp, dwn, ohw,
             w1a, w1b, w1c, w1d, b1, w2, b2, w3, b3, u, gs):
    sigp = 1.0 - jnp.exp(-litp[0])
    sign_ = 1.0 - jnp.exp(-litn[0])
    spv = sp[...]
    snv = sn[...]
    vg = (-spv[:, :FM] * sigp + snv[:, :FM] * sign_) * vdw[...]
    vlp = spv[:, FM:] * dwp[...]
    vln = snv[:, FM:] * dwn[...]
    h = jnp.maximum(vg @ w1a[...] + v[...] @ w1b[...]
                    + vlp @ w1c[...] + vln @ w1d[...] + b1[...], 0.0)
    h = jnp.maximum(h @ w2[...] + b2[...], 0.0)
    uu = h @ w3[...] + b3[...]
    u[...] = uu

    @pl.when(pl.program_id(0) == 0)
    def _():
        gs[...] = jnp.zeros_like(gs)

    gs[...] += lax.dot_general(ohw[...], uu, (((0,), (0,)), ((), ())))


def _k3(v, lit, s, vdw, dwp, dwn, ohw, w1a, w1b, w1c, w1d, b1, w2, b2, w3, b3):
    lit_spec_p = pl.BlockSpec((1, BT, FM), lambda i: (0, i, 0))
    lit_spec_n = pl.BlockSpec((1, BT, FM), lambda i: (1, i, 0))
    sp_spec = pl.BlockSpec((BT, FM + QM), lambda i: (i, 0))
    sn_spec = pl.BlockSpec((BT, FM + QM), lambda i: (i + NBV, 0))
    col1 = pl.BlockSpec((BT, 1), lambda i: (i, 0))
    return pl.pallas_call(
        _k3_body,
        grid=(NBV,),
        in_specs=[_rows(FM), lit_spec_p, lit_spec_n, sp_spec, sn_spec,
                  col1, col1, col1,
                  pl.BlockSpec((BT, N_GRAPHS), lambda i: (i, 0)),
                  _full(w1a), _full(w1b), _full(w1c), _full(w1d),
                  _full(b1), _full(w2), _full(b2), _full(w3), _full(b3)],
        out_specs=[_rows(FM), pl.BlockSpec((N_GRAPHS, FM), lambda i: (0, 0))],
        out_shape=[jax.ShapeDtypeStruct((N_VARS, FM), jnp.float32),
                   jax.ShapeDtypeStruct((N_GRAPHS, FM), jnp.float32)],
    )(v, lit, lit, s, s, vdw, dwp, dwn, ohw,
      w1a, w1b, w1c, w1d, b1, w2, b2, w3, b3)


def _k4_body(c, fn, w1, b1, w2, b2, out):
    h = jnp.maximum(c[...] @ w1[...] + b1[...], 0.0)
    lg = h @ w2[...] + b2[...]
    out[...] = jax.nn.sigmoid(lg + fn[...])


def _k4(c, fn, w1, b1, w2, b2):
    col1 = pl.BlockSpec((BT, 1), lambda i: (i, 0))
    return pl.pallas_call(
        _k4_body,
        grid=(NBC,),
        in_specs=[_rows(FM), col1, _full(w1), _full(b1), _full(w2), _full(b2)],
        out_specs=col1,
        out_shape=jax.ShapeDtypeStruct((N_CLAUSES, 1), jnp.float32),
    )(c, fn, w1, b1, w2, b2)


def _prep_pass(sort_key, payload, ch, n_chunks, kmax):
    perm = jnp.argsort(sort_key)
    key_s = sort_key[perm].astype(jnp.int32)
    pay_s = payload[perm].astype(jnp.int32)
    bounds = jnp.arange(n_chunks + 1, dtype=jnp.int32) * ch
    starts = jnp.searchsorted(key_s, bounds).astype(jnp.int32)
    counts = starts[1:] - starts[:-1]
    pcounts = ((counts + EB - 1) // EB) * EB
    pstarts = jnp.concatenate(
        [jnp.zeros((1,), jnp.int32), jnp.cumsum(pcounts).astype(jnp.int32)])
    chunk_id = key_s // ch
    pos = pstarts[chunk_id] + (jnp.arange(NNZ, dtype=jnp.int32)
                               - starts[chunk_id])
    e_pad = NNZ + n_chunks * EB
    slot = ch + 1
    slot_of = lambda c: ((c % NTILES) // 2) * slot
    pos_arr = jnp.arange(e_pad, dtype=jnp.int32)
    chunk_of_pos = jnp.clip(
        jnp.searchsorted(pstarts, pos_arr, side="right").astype(jnp.int32) - 1,
        0, n_chunks - 1)
    eidx = jnp.zeros((e_pad,), jnp.int32).at[pos].set(pay_s)
    lidx = (slot_of(chunk_of_pos) + ch).at[pos].set(
        slot_of(chunk_id) + key_s - chunk_id * ch)
    cid = (jnp.arange(NTILES, dtype=jnp.int32)[:, None]
           + NTILES * jnp.arange(kmax, dtype=jnp.int32)[None, :])
    valid = cid < n_chunks
    cidc = jnp.minimum(cid, n_chunks - 1)
    ms = jnp.where(valid, pstarts[cidc], 0)
    me = jnp.where(valid, pstarts[cidc] + pcounts[cidc], 0)
    meta = jnp.zeros((NTILES, 16), jnp.int32)
    meta = meta.at[:, 0:2 * kmax:2].set(ms)
    meta = meta.at[:, 1:2 * kmax:2].set(me)
    return eidx, lidx, meta


def _onehot_w(gids, n):
    cnt = jnp.zeros((N_GRAPHS,), jnp.float32).at[gids].add(1.0)
    wg = 1.0 / jnp.maximum(cnt, 1.0)
    oh = (gids[:, None] == jnp.arange(N_GRAPHS)[None, :]).astype(jnp.float32)
    return oh * wg[gids][:, None]


def kernel(adj_rows, adj_cols, variables_graph_ids, clauses_graph_ids,
           vq_w1, vq_b1, vq_w2, vq_b2, cm_w1, cm_b1, cm_w2, cm_b2,
           ug_w1, ug_b1, ug_w2, ug_b2, ug_w3, ug_b3, co_w1, co_b1,
           co_w2, co_b2):
    adj_rows = adj_rows.astype(jnp.int32)
    adj_cols = adj_cols.astype(jnp.int32)

    eidx1, lidx1, meta1 = _prep_pass(adj_cols, adj_rows, CH1, NCH1, KMAX1)
    eidx2, lidx2, meta2 = _prep_pass(adj_rows, adj_cols, CH2, NCH2, KMAX2)
    zeros1 = jnp.zeros((CH1 + 1, FM), jnp.float32)
    zeros2 = jnp.zeros((CH2 + 1, FM + QM), jnp.float32)

    deg = jnp.zeros((LITS,), jnp.float32).at[adj_rows].add(1.0)
    dw = lax.rsqrt(jnp.maximum(deg, 1.0))[:, None]
    vdw = 4.0 * lax.rsqrt(jnp.maximum(deg[:N_VARS] + deg[N_VARS:], 1.0))[:, None]
    dwp, dwn = dw[:N_VARS], dw[N_VARS:]

    ohwv = _onehot_w(variables_graph_ids, N_VARS)
    ohwc = _onehot_w(clauses_graph_ids, N_CLAUSES)

    noises = [jax.random.normal(jax.random.fold_in(jax.random.key(1), r),
                                (N_VARS, 4), jnp.float32) for r in range(ROUNDS)]
    final_noise = jax.random.normal(
        jax.random.fold_in(jax.random.key(1), 12345), (N_CLAUSES, 1), jnp.float32)

    vq_w1a, vq_w1b = vq_w1[:FM], vq_w1[FM:]
    cm_w1a, cm_w1b = cm_w1[:FM], cm_w1[FM:]
    ug_w1a = ug_w1[:FM]
    ug_w1b = ug_w1[FM:2 * FM]
    ug_w1c = ug_w1[2 * FM:3 * FM]
    ug_w1d = ug_w1[3 * FM:]
    vq_b1r, vq_b2r = vq_b1[None, :], vq_b2[None, :]
    cm_b1r, cm_b2r = cm_b1[None, :], cm_b2[None, :]
    ug_b1r, ug_b2r, ug_b3r = ug_b1[None, :], ug_b2[None, :], ug_b3[None, :]
    co_b1r, co_b2r = co_b1[None, :], co_b2[None, :]

    variables = jnp.ones((N_VARS, FM), jnp.float32)
    clauses = jnp.ones((N_CLAUSES, FM), jnp.float32)

    _seg1 = _seg_sum_cached(FM, CH1, NCH1, KMAX1)
    _seg2 = _seg_sum_cached(FM + QM, CH2, NCH2, KMAX2)

    for r in range(ROUNDS):
        last = r == ROUNDS - 1
        lit = _k1(variables, noises[r], vq_w1a, vq_w1b, vq_b1r, vq_w2, vq_b2r)
        lit2 = lit.reshape(LITS, FM)
        cval = _seg1(lit2, eidx1, lidx1, meta1, zeros1)[:N_CLAUSES]
        tab, cd2, gsc = _k2(cval, clauses, ohwc,
                            cm_w1a, cm_w1b, cm_b1r, cm_w2, cm_b2r)
        clauses = _pn(cd2, clauses, ohwc, gsc, 1.0 if last else 0.2)
        if not last:
            s = _seg2(tab, eidx2, lidx2, meta2, zeros2)[:LITS]
            u, gsv = _k3(variables, lit, s, vdw, dwp, dwn, ohwv,
                         ug_w1a, ug_w1b, ug_w1c, ug_w1d, ug_b1r,
                         ug_w2, ug_b2r, ug_w3, ug_b3r)
            variables = _pn(u, variables, ohwv, gsv, 1.0)

    out = _k4(clauses, final_noise, co_w1, co_b1r, co_w2, co_b2r)
    return out.reshape(N_CLAUSES)
